# baseline (speedup 1.0000x reference)
import functools

import jax
import jax.numpy as jnp
from jax import lax
from jax.experimental import pallas as pl
from jax.experimental.pallas import tpu as pltpu
from jax.experimental.pallas import tpu_sc as plsc

B, S, H, FFN, E, C = 2, 2048, 1024, 4096, 8, 640
N = B * S
EROWS = 2 * C + 64
R_TOT = E * EROWS
NHALF = R_TOT // 2
ZROW = 2 * C
SENT = ZROW + 16
TB = 256
NBLK = N // TB
SPB = S // TB
FB = 1024
NJ = FFN // FB
NW = 32
RPW = NHALF // NW
TPW = N // NW
CH = 16


def _sc_mesh():
    return plsc.VectorSubcoreMesh(
        core_axis_name="c", subcore_axis_name="s", num_cores=2, num_subcores=16)


_SC_PARAMS = pltpu.CompilerParams(needs_layout_passes=False)


def _routing_body(x_ref, gw_ref, d1_ref, d2_ref, g1_ref, g2_ref,
                  p1_ref, p2_ref, carry_ref):
    i = pl.program_id(0)
    b = i // SPB

    @pl.when(lax.rem(i, SPB) == 0)
    def _():
        carry_ref[...] = jnp.zeros_like(carry_ref)

    x = x_ref[...]
    logits = lax.dot_general(x.astype(jnp.bfloat16),
                             gw_ref[...].astype(jnp.bfloat16),
                             (((1,), (1,)), ((), ())),
                             preferred_element_type=jnp.float32)
    m = jnp.max(logits, axis=1, keepdims=True)
    ex = jnp.exp(logits - m)
    probs = ex / jnp.sum(ex, axis=1, keepdims=True)

    ie = lax.broadcasted_iota(jnp.int32, (TB, E), 1)
    p1 = jnp.max(probs, axis=1)
    e1 = jnp.min(jnp.where(probs == p1[:, None], ie, E), axis=1)
    probs_m = jnp.where(ie == e1[:, None], -1.0, probs)
    p2 = jnp.max(probs_m, axis=1)
    e2 = jnp.min(jnp.where(probs_m == p2[:, None], ie, E), axis=1)

    oh1 = (ie == e1[:, None]).astype(jnp.float32)
    oh2 = (ie == e2[:, None]).astype(jnp.float32)
    a = oh1 + oh2
    ri = lax.broadcasted_iota(jnp.int32, (TB, TB), 0)
    ci = lax.broadcasted_iota(jnp.int32, (TB, TB), 1)
    tri = (ci < ri).astype(jnp.float32)
    prefix = lax.dot_general(tri, a, (((1,), (0,)), ((), ())),
                             precision=lax.Precision.HIGHEST,
                             preferred_element_type=jnp.float32)
    cp = carry_ref[...] + prefix
    pos1 = jnp.sum(cp * oh1, axis=1).astype(jnp.int32) + 1
    pos2 = jnp.sum(cp * oh2, axis=1).astype(jnp.int32) + 1
    carry_ref[...] += jnp.sum(a, axis=0, keepdims=True)

    slot1 = e1 * EROWS + b * C + pos1 - 1
    slot2 = e2 * EROWS + b * C + pos2 - 1
    keep1 = pos1 <= C
    keep2 = pos2 <= C
    d1_ref[0, 0, :] = jnp.where(keep1, slot1, SENT)
    d2_ref[0, 0, :] = jnp.where(keep2, slot2, SENT)
    g1_ref[0, 0, :] = jnp.where(keep1, slot1, ZROW)
    g2_ref[0, 0, :] = jnp.where(keep2, slot2, ZROW)
    p1_ref[0, 0, :] = p1
    p2_ref[0, 0, :] = p2


def _routing(x, gate_w):
    outs = pl.pallas_call(
        _routing_body,
        grid=(NBLK,),
        in_specs=[pl.BlockSpec((TB, H), lambda i: (i, 0)),
                  pl.BlockSpec((E, H), lambda i: (0, 0))],
        out_specs=[pl.BlockSpec((1, 1, TB), lambda i: (i, 0, 0))] * 6,
        out_shape=[jax.ShapeDtypeStruct((NBLK, 1, TB), jnp.int32)] * 4
        + [jax.ShapeDtypeStruct((NBLK, 1, TB), jnp.float32)] * 2,
        scratch_shapes=[pltpu.VMEM((1, E), jnp.float32)],
    )(x, gate_w)
    return [o.reshape(N) for o in outs]


def _sc_maps(d1, d2, p1, p2):
    @functools.partial(
        pl.kernel,
        out_type=(jax.ShapeDtypeStruct((R_TOT,), jnp.int32),
                  jax.ShapeDtypeStruct((R_TOT,), jnp.float32)),
        mesh=_sc_mesh(),
        compiler_params=_SC_PARAMS,
        scratch_types=[pltpu.VMEM((N,), jnp.int32),
                       pltpu.VMEM((N,), jnp.int32),
                       pltpu.VMEM((N,), jnp.float32),
                       pltpu.VMEM((N,), jnp.float32),
                       pltpu.VMEM((R_TOT,), jnp.int32),
                       pltpu.VMEM((R_TOT,), jnp.float32)],
    )
    def k(d1_hbm, d2_hbm, p1_hbm, p2_hbm, smap_hbm, wmap_hbm,
          d1_v, d2_v, p1_v, p2_v, smap_v, wmap_v):
        wid = lax.axis_index("s") * 2 + lax.axis_index("c")

        @pl.when(wid == 0)
        def _():
            pltpu.sync_copy(d1_hbm, d1_v)
            pltpu.sync_copy(d2_hbm, d2_v)
            pltpu.sync_copy(p1_hbm, p1_v)
            pltpu.sync_copy(p2_hbm, p2_v)

            def init(i, c):
                sl = pl.ds(i * 16, 16)
                smap_v[sl] = jnp.zeros((16,), jnp.int32)
                wmap_v[sl] = jnp.zeros((16,), jnp.float32)
                return c
            lax.fori_loop(0, R_TOT // 16, init, 0)

            def body(i, c):
                sl = pl.ds(i * 16, 16)
                tok = lax.iota(jnp.int32, 16) + i * 16
                d1c = d1_v[sl]
                plsc.store_scatter(smap_v, [d1c], tok, mask=d1c != SENT)
                plsc.store_scatter(wmap_v, [d1c], p1_v[sl], mask=d1c != SENT)
                d2c = d2_v[sl]
                plsc.store_scatter(smap_v, [d2c], tok, mask=d2c != SENT)
                plsc.store_scatter(wmap_v, [d2c], p2_v[sl], mask=d2c != SENT)
                return c
            lax.fori_loop(0, N // 16, body, 0)
            pltpu.sync_copy(smap_v, smap_hbm)
            pltpu.sync_copy(wmap_v, wmap_hbm)

    return k(d1, d2, p1, p2)


def _sc_dispatch(x, smap, half):
    chunks = [(o, 16) for o in range(0, 160, 16)] + [(160, 8)]
    nch = len(chunks)
    nbuf = 4

    @functools.partial(
        pl.kernel,
        out_type=jax.ShapeDtypeStruct((NHALF, H), jnp.float32),
        mesh=_sc_mesh(),
        compiler_params=_SC_PARAMS,
        scratch_types=[pltpu.VMEM((RPW,), jnp.int32)]
        + [pltpu.VMEM((16, H), jnp.float32)] * 4
        + [pltpu.SemaphoreType.DMA] * 8,
    )
    def k(x_hbm, smap_hbm, disp_hbm, idx_v, b0, b1, b2, b3,
          g0, g1s, g2s, g3s, s0, s1, s2, s3):
        wid = lax.axis_index("s") * 2 + lax.axis_index("c")
        base = wid * RPW
        bufs = (b0, b1, b2, b3)
        gsems = (g0, g1s, g2s, g3s)
        ssems = (s0, s1, s2, s3)
        pltpu.sync_copy(smap_hbm.at[pl.ds(half * NHALF + base, RPW)], idx_v)

        def gather(c):
            off, sz = chunks[c]
            return pltpu.make_async_copy(
                x_hbm.at[idx_v.at[pl.ds(off, sz)]],
                bufs[c % nbuf].at[pl.ds(0, sz)], gsems[c % nbuf])

        def store(c):
            off, sz = chunks[c]
            return pltpu.make_async_copy(
                bufs[c % nbuf].at[pl.ds(0, sz)],
                disp_hbm.at[pl.ds(base + off, sz)], ssems[c % nbuf])

        gd = {}
        for c in range(nbuf):
            gd[c] = gather(c)
            gd[c].start()
        sd = {}
        for c in range(nch):
            if c >= 1 and c + nbuf - 1 < nch:
                sd[c - 1].wait()
                gd[c + nbuf - 1] = gather(c + nbuf - 1)
                gd[c + nbuf - 1].start()
            gd[c].wait()
            sd[c] = store(c)
            sd[c].start()
        for c in range(max(0, nch - nbuf), nch):
            sd[c].wait()

    return k(x, smap)


def _mlp_body(x_ref, w1_ref, w3_ref, w2_ref, wt_ref, out_ref):
    j = pl.program_id(1)
    x = x_ref[...].astype(jnp.bfloat16)
    h1 = jnp.dot(x, w1_ref[0].astype(jnp.bfloat16),
                 preferred_element_type=jnp.float32)
    h3 = jnp.dot(x, w3_ref[0].astype(jnp.bfloat16),
                 preferred_element_type=jnp.float32)
    hh = (h1 / (1.0 + jnp.exp(-h1))) * h3
    contrib = jnp.dot(hh.astype(jnp.bfloat16), w2_ref[0].astype(jnp.bfloat16),
                      preferred_element_type=jnp.float32)

    @pl.when(j == 0)
    def _():
        out_ref[...] = contrib

    @pl.when(jnp.logical_and(j > 0, j < NJ - 1))
    def _():
        out_ref[...] += contrib

    @pl.when(j == NJ - 1)
    def _():
        out_ref[...] = (out_ref[...] + contrib) * wt_ref[...]


def _mlp(disp, w1, w3, w2, wt, off):
    return pl.pallas_call(
        _mlp_body,
        grid=(E // 2, NJ),
        in_specs=[pl.BlockSpec((EROWS, H), lambda e, j: (e, 0)),
                  pl.BlockSpec((1, H, FB), lambda e, j: (e + off, 0, j)),
                  pl.BlockSpec((1, H, FB), lambda e, j: (e + off, 0, j)),
                  pl.BlockSpec((1, FB, H), lambda e, j: (e + off, j, 0)),
                  pl.BlockSpec((EROWS, 1), lambda e, j: (e + off, 0))],
        out_specs=pl.BlockSpec((EROWS, H), lambda e, j: (e, 0)),
        out_shape=jax.ShapeDtypeStruct((NHALF, H), jnp.float32),
    )(disp, w1, w3, w2, wt)


def _sc_combine(eo, g1, g2):
    @functools.partial(
        pl.kernel,
        out_type=jax.ShapeDtypeStruct((N, H), jnp.float32),
        mesh=_sc_mesh(),
        compiler_params=_SC_PARAMS,
        scratch_types=[pltpu.VMEM((TPW,), jnp.int32),
                       pltpu.VMEM((TPW,), jnp.int32),
                       pltpu.VMEM((CH, H), jnp.float32),
                       pltpu.VMEM((CH, H), jnp.float32),
                       pltpu.VMEM((CH, H), jnp.float32),
                       pltpu.VMEM((CH, H), jnp.float32),
                       pltpu.SemaphoreType.DMA,
                       pltpu.SemaphoreType.DMA,
                       pltpu.SemaphoreType.DMA,
                       pltpu.SemaphoreType.DMA,
                       pltpu.SemaphoreType.DMA,
                       pltpu.SemaphoreType.DMA],
    )
    def k(eo_hbm, g1_hbm, g2_hbm, out_hbm, g1_v, g2_v,
          a0, a1, c0, c1, sa0, sa1, sc0, sc1, so0, so1):
        wid = lax.axis_index("s") * 2 + lax.axis_index("c")
        base = wid * TPW
        abuf = (a0, a1)
        bbuf = (c0, c1)
        asem = (sa0, sa1)
        bsem = (sc0, sc1)
        osem = (so0, so1)
        nch = TPW // CH
        pltpu.sync_copy(g1_hbm.at[pl.ds(base, TPW)], g1_v)
        pltpu.sync_copy(g2_hbm.at[pl.ds(base, TPW)], g2_v)

        def gathers(c):
            p = c % 2
            d1 = pltpu.make_async_copy(
                eo_hbm.at[g1_v.at[pl.ds(c * CH, CH)]], abuf[p], asem[p])
            d2 = pltpu.make_async_copy(
                eo_hbm.at[g2_v.at[pl.ds(c * CH, CH)]], bbuf[p], bsem[p])
            d1.start()
            d2.start()
            return d1, d2

        def outstore(c):
            p = c % 2
            return pltpu.make_async_copy(
                abuf[p], out_hbm.at[pl.ds(base + c * CH, CH)], osem[p])

        gd = {0: gathers(0)}
        od = {}
        for c in range(nch):
            if c + 1 < nch:
                if c >= 1:
                    od[c - 1].wait()
                gd[c + 1] = gathers(c + 1)
            d1, d2 = gd[c]
            d1.wait()
            d2.wait()
            p = c % 2

            def add(i, cc):
                r = i // 8
                col0 = lax.rem(i, 8) * 128
                for u in range(8):
                    sl = pl.ds(col0 + u * 16, 16)
                    abuf[p][r, sl] = abuf[p][r, sl] + bbuf[p][r, sl]
                return cc
            lax.fori_loop(0, CH * 8, add, 0)
            od[c] = outstore(c)
            od[c].start()
        od[nch - 2].wait()
        od[nch - 1].wait()

    return k(eo, g1, g2)


def kernel(hidden_states, gate_w, w1, w2, w3):
    x = hidden_states.reshape(N, H)
    d1, d2, g1, g2, p1, p2 = _routing(x, gate_w)
    smap, wmap = _sc_maps(d1, d2, p1, p2)
    wt = wmap.reshape(R_TOT, 1)
    disp_a = _sc_dispatch(x, smap, 0)
    disp_b = _sc_dispatch(x, smap, 1)
    eo_a = _mlp(disp_a, w1, w3, w2, wt, 0)
    eo_b = _mlp(disp_b, w1, w3, w2, wt, E // 2)
    eo = jnp.concatenate([eo_a, eo_b], axis=0)
    out = _sc_combine(eo, g1, g2)
    return out.reshape(B, S, H)


# speedup vs baseline: 1.0405x; 1.0405x over previous
import functools

import jax
import jax.numpy as jnp
from jax import lax
from jax.experimental import pallas as pl
from jax.experimental.pallas import tpu as pltpu
from jax.experimental.pallas import tpu_sc as plsc

B, S, H, FFN, E, C = 2, 2048, 1024, 4096, 8, 640
N = B * S
EROWS = 2 * C + 64
R_TOT = E * EROWS
NHALF = R_TOT // 2
ZROW = 2 * C
SENT = ZROW + 16
TB = 256
NBLK = N // TB
SPB = S // TB
FB = 1024
NJ = FFN // FB
NW = 32
RPW = R_TOT // NW
TPW = N // NW
CH = 16


def _sc_mesh():
    return plsc.VectorSubcoreMesh(
        core_axis_name="c", subcore_axis_name="s", num_cores=2, num_subcores=16)


_SC_PARAMS = pltpu.CompilerParams(needs_layout_passes=False)


def _routing_body(x_ref, gw_ref, d1_ref, d2_ref, g1_ref, g2_ref,
                  p1_ref, p2_ref, carry_ref):
    i = pl.program_id(0)
    b = i // SPB

    @pl.when(lax.rem(i, SPB) == 0)
    def _():
        carry_ref[...] = jnp.zeros_like(carry_ref)

    x = x_ref[...]
    logits = lax.dot_general(x.astype(jnp.bfloat16),
                             gw_ref[...].astype(jnp.bfloat16),
                             (((1,), (1,)), ((), ())),
                             preferred_element_type=jnp.float32)
    m = jnp.max(logits, axis=1, keepdims=True)
    ex = jnp.exp(logits - m)
    probs = ex / jnp.sum(ex, axis=1, keepdims=True)

    ie = lax.broadcasted_iota(jnp.int32, (TB, E), 1)
    p1 = jnp.max(probs, axis=1)
    e1 = jnp.min(jnp.where(probs == p1[:, None], ie, E), axis=1)
    probs_m = jnp.where(ie == e1[:, None], -1.0, probs)
    p2 = jnp.max(probs_m, axis=1)
    e2 = jnp.min(jnp.where(probs_m == p2[:, None], ie, E), axis=1)

    oh1 = (ie == e1[:, None]).astype(jnp.float32)
    oh2 = (ie == e2[:, None]).astype(jnp.float32)
    a = oh1 + oh2
    ri = lax.broadcasted_iota(jnp.int32, (TB, TB), 0)
    ci = lax.broadcasted_iota(jnp.int32, (TB, TB), 1)
    tri = (ci < ri).astype(jnp.float32)
    prefix = lax.dot_general(tri, a, (((1,), (0,)), ((), ())),
                             precision=lax.Precision.HIGHEST,
                             preferred_element_type=jnp.float32)
    cp = carry_ref[...] + prefix
    pos1 = jnp.sum(cp * oh1, axis=1).astype(jnp.int32) + 1
    pos2 = jnp.sum(cp * oh2, axis=1).astype(jnp.int32) + 1
    carry_ref[...] += jnp.sum(a, axis=0, keepdims=True)

    slot1 = e1 * EROWS + b * C + pos1 - 1
    slot2 = e2 * EROWS + b * C + pos2 - 1
    keep1 = pos1 <= C
    keep2 = pos2 <= C
    d1_ref[0, 0, :] = jnp.where(keep1, slot1, SENT)
    d2_ref[0, 0, :] = jnp.where(keep2, slot2, SENT)
    g1_ref[0, 0, :] = jnp.where(keep1, slot1, ZROW)
    g2_ref[0, 0, :] = jnp.where(keep2, slot2, ZROW)
    p1_ref[0, 0, :] = p1
    p2_ref[0, 0, :] = p2


def _routing(x, gate_w):
    outs = pl.pallas_call(
        _routing_body,
        grid=(NBLK,),
        in_specs=[pl.BlockSpec((TB, H), lambda i: (i, 0)),
                  pl.BlockSpec((E, H), lambda i: (0, 0))],
        out_specs=[pl.BlockSpec((1, 1, TB), lambda i: (i, 0, 0))] * 6,
        out_shape=[jax.ShapeDtypeStruct((NBLK, 1, TB), jnp.int32)] * 4
        + [jax.ShapeDtypeStruct((NBLK, 1, TB), jnp.float32)] * 2,
        scratch_shapes=[pltpu.VMEM((1, E), jnp.float32)],
    )(x, gate_w)
    return [o.reshape(N) for o in outs]


def _sc_maps(d1, d2, p1, p2):
    @functools.partial(
        pl.kernel,
        out_type=(jax.ShapeDtypeStruct((R_TOT,), jnp.int32),
                  jax.ShapeDtypeStruct((R_TOT,), jnp.float32)),
        mesh=_sc_mesh(),
        compiler_params=_SC_PARAMS,
        scratch_types=[pltpu.VMEM((N,), jnp.int32),
                       pltpu.VMEM((N,), jnp.int32),
                       pltpu.VMEM((N,), jnp.float32),
                       pltpu.VMEM((N,), jnp.float32),
                       pltpu.VMEM((R_TOT,), jnp.int32),
                       pltpu.VMEM((R_TOT,), jnp.float32)],
    )
    def k(d1_hbm, d2_hbm, p1_hbm, p2_hbm, smap_hbm, wmap_hbm,
          d1_v, d2_v, p1_v, p2_v, smap_v, wmap_v):
        wid = lax.axis_index("s") * 2 + lax.axis_index("c")

        @pl.when(wid == 0)
        def _():
            pltpu.sync_copy(d1_hbm, d1_v)
            pltpu.sync_copy(d2_hbm, d2_v)
            pltpu.sync_copy(p1_hbm, p1_v)
            pltpu.sync_copy(p2_hbm, p2_v)

            def init(i, c):
                sl = pl.ds(i * 16, 16)
                smap_v[sl] = jnp.zeros((16,), jnp.int32)
                wmap_v[sl] = jnp.zeros((16,), jnp.float32)
                return c
            lax.fori_loop(0, R_TOT // 16, init, 0)

            def body(i, c):
                sl = pl.ds(i * 16, 16)
                tok = lax.iota(jnp.int32, 16) + i * 16
                d1c = d1_v[sl]
                plsc.store_scatter(smap_v, [d1c], tok, mask=d1c != SENT)
                plsc.store_scatter(wmap_v, [d1c], p1_v[sl], mask=d1c != SENT)
                d2c = d2_v[sl]
                plsc.store_scatter(smap_v, [d2c], tok, mask=d2c != SENT)
                plsc.store_scatter(wmap_v, [d2c], p2_v[sl], mask=d2c != SENT)
                return c
            lax.fori_loop(0, N // 16, body, 0)
            pltpu.sync_copy(smap_v, smap_hbm)
            pltpu.sync_copy(wmap_v, wmap_hbm)

    return k(d1, d2, p1, p2)


def _sc_dispatch(x, smap):
    chunks = [(o, 16) for o in range(0, RPW, 16)]
    nch = len(chunks)
    nbuf = 4

    @functools.partial(
        pl.kernel,
        out_type=jax.ShapeDtypeStruct((R_TOT, H), jnp.float32),
        mesh=_sc_mesh(),
        compiler_params=_SC_PARAMS,
        scratch_types=[pltpu.VMEM((RPW,), jnp.int32)]
        + [pltpu.VMEM((16, H), jnp.float32)] * 4
        + [pltpu.SemaphoreType.DMA] * 8,
    )
    def k(x_hbm, smap_hbm, disp_hbm, idx_v, b0, b1, b2, b3,
          g0, g1s, g2s, g3s, s0, s1, s2, s3):
        wid = lax.axis_index("s") * 2 + lax.axis_index("c")
        base = wid * RPW
        bufs = (b0, b1, b2, b3)
        gsems = (g0, g1s, g2s, g3s)
        ssems = (s0, s1, s2, s3)
        pltpu.sync_copy(smap_hbm.at[pl.ds(base, RPW)], idx_v)

        def gather(c):
            off, sz = chunks[c]
            return pltpu.make_async_copy(
                x_hbm.at[idx_v.at[pl.ds(off, sz)]],
                bufs[c % nbuf].at[pl.ds(0, sz)], gsems[c % nbuf])

        def store(c):
            off, sz = chunks[c]
            return pltpu.make_async_copy(
                bufs[c % nbuf].at[pl.ds(0, sz)],
                disp_hbm.at[pl.ds(base + off, sz)], ssems[c % nbuf])

        gd = {}
        for c in range(nbuf):
            gd[c] = gather(c)
            gd[c].start()
        sd = {}
        for c in range(nch):
            if c >= 1 and c + nbuf - 1 < nch:
                sd[c - 1].wait()
                gd[c + nbuf - 1] = gather(c + nbuf - 1)
                gd[c + nbuf - 1].start()
            gd[c].wait()
            sd[c] = store(c)
            sd[c].start()
        for c in range(max(0, nch - nbuf), nch):
            sd[c].wait()

    return k(x, smap)


def _mlp_body(x_ref, w1_ref, w3_ref, w2_ref, wt_ref, out_ref):
    j = pl.program_id(1)
    x = x_ref[...].astype(jnp.bfloat16)
    h1 = jnp.dot(x, w1_ref[0].astype(jnp.bfloat16),
                 preferred_element_type=jnp.float32)
    h3 = jnp.dot(x, w3_ref[0].astype(jnp.bfloat16),
                 preferred_element_type=jnp.float32)
    hh = (h1 / (1.0 + jnp.exp(-h1))) * h3
    contrib = jnp.dot(hh.astype(jnp.bfloat16), w2_ref[0].astype(jnp.bfloat16),
                      preferred_element_type=jnp.float32)

    @pl.when(j == 0)
    def _():
        out_ref[...] = contrib

    @pl.when(jnp.logical_and(j > 0, j < NJ - 1))
    def _():
        out_ref[...] += contrib

    @pl.when(j == NJ - 1)
    def _():
        out_ref[...] = (out_ref[...] + contrib) * wt_ref[...]


def _mlp(disp, w1, w3, w2, wt):
    return pl.pallas_call(
        _mlp_body,
        grid=(E, NJ),
        in_specs=[pl.BlockSpec((EROWS, H), lambda e, j: (e, 0)),
                  pl.BlockSpec((1, H, FB), lambda e, j: (e, 0, j)),
                  pl.BlockSpec((1, H, FB), lambda e, j: (e, 0, j)),
                  pl.BlockSpec((1, FB, H), lambda e, j: (e, j, 0)),
                  pl.BlockSpec((EROWS, 1), lambda e, j: (e, 0))],
        out_specs=pl.BlockSpec((EROWS, H), lambda e, j: (e, 0)),
        out_shape=jax.ShapeDtypeStruct((R_TOT, H), jnp.float32),
    )(disp, w1, w3, w2, wt)


def _sc_combine(eo, g1, g2):
    @functools.partial(
        pl.kernel,
        out_type=jax.ShapeDtypeStruct((N, H), jnp.float32),
        mesh=_sc_mesh(),
        compiler_params=_SC_PARAMS,
        scratch_types=[pltpu.VMEM((TPW,), jnp.int32),
                       pltpu.VMEM((TPW,), jnp.int32),
                       pltpu.VMEM((CH, H), jnp.float32),
                       pltpu.VMEM((CH, H), jnp.float32),
                       pltpu.VMEM((CH, H), jnp.float32),
                       pltpu.VMEM((CH, H), jnp.float32),
                       pltpu.SemaphoreType.DMA,
                       pltpu.SemaphoreType.DMA,
                       pltpu.SemaphoreType.DMA,
                       pltpu.SemaphoreType.DMA,
                       pltpu.SemaphoreType.DMA,
                       pltpu.SemaphoreType.DMA],
    )
    def k(eo_hbm, g1_hbm, g2_hbm, out_hbm, g1_v, g2_v,
          a0, a1, c0, c1, sa0, sa1, sc0, sc1, so0, so1):
        wid = lax.axis_index("s") * 2 + lax.axis_index("c")
        base = wid * TPW
        abuf = (a0, a1)
        bbuf = (c0, c1)
        asem = (sa0, sa1)
        bsem = (sc0, sc1)
        osem = (so0, so1)
        nch = TPW // CH
        pltpu.sync_copy(g1_hbm.at[pl.ds(base, TPW)], g1_v)
        pltpu.sync_copy(g2_hbm.at[pl.ds(base, TPW)], g2_v)

        def gathers(c):
            p = c % 2
            d1 = pltpu.make_async_copy(
                eo_hbm.at[g1_v.at[pl.ds(c * CH, CH)]], abuf[p], asem[p])
            d2 = pltpu.make_async_copy(
                eo_hbm.at[g2_v.at[pl.ds(c * CH, CH)]], bbuf[p], bsem[p])
            d1.start()
            d2.start()
            return d1, d2

        def outstore(c):
            p = c % 2
            return pltpu.make_async_copy(
                abuf[p], out_hbm.at[pl.ds(base + c * CH, CH)], osem[p])

        gd = {0: gathers(0)}
        od = {}
        for c in range(nch):
            if c + 1 < nch:
                if c >= 1:
                    od[c - 1].wait()
                gd[c + 1] = gathers(c + 1)
            d1, d2 = gd[c]
            d1.wait()
            d2.wait()
            p = c % 2

            def add(i, cc):
                r = i // 8
                col0 = lax.rem(i, 8) * 128
                for u in range(8):
                    sl = pl.ds(col0 + u * 16, 16)
                    abuf[p][r, sl] = abuf[p][r, sl] + bbuf[p][r, sl]
                return cc
            lax.fori_loop(0, CH * 8, add, 0)
            od[c] = outstore(c)
            od[c].start()
        od[nch - 2].wait()
        od[nch - 1].wait()

    return k(eo, g1, g2)


def kernel(hidden_states, gate_w, w1, w2, w3):
    x = hidden_states.reshape(N, H)
    d1, d2, g1, g2, p1, p2 = _routing(x, gate_w)
    smap, wmap = _sc_maps(d1, d2, p1, p2)
    disp = _sc_dispatch(x, smap)
    eo = _mlp(disp, w1, w3, w2, wmap.reshape(R_TOT, 1))
    out = _sc_combine(eo, g1, g2)
    return out.reshape(B, S, H)


# speedup vs baseline: 1.0821x; 1.0399x over previous
import functools

import jax
import jax.numpy as jnp
from jax import lax
from jax.experimental import pallas as pl
from jax.experimental.pallas import tpu as pltpu
from jax.experimental.pallas import tpu_sc as plsc

B, S, H, FFN, E, C = 2, 2048, 1024, 4096, 8, 640
N = B * S
EROWS = 2 * C + 32
R_TOT = E * EROWS
ZROW = 2 * C
SENT = ZROW + 16
TB = 256
NBLK = N // TB
SPB = S // TB
FB = 1024
NJ = FFN // FB
NW = 32
RPW = R_TOT // NW
TPW = N // NW
CH = 16


def _sc_mesh():
    return plsc.VectorSubcoreMesh(
        core_axis_name="c", subcore_axis_name="s", num_cores=2, num_subcores=16)


_SC_PARAMS = pltpu.CompilerParams(needs_layout_passes=False)


def _routing_body(x_ref, gw_ref, d1_ref, d2_ref, g1_ref, g2_ref,
                  p1_ref, p2_ref, carry_ref):
    i = pl.program_id(0)
    b = i // SPB

    @pl.when(lax.rem(i, SPB) == 0)
    def _():
        carry_ref[...] = jnp.zeros_like(carry_ref)

    x = x_ref[...]
    logits = lax.dot_general(x.astype(jnp.bfloat16),
                             gw_ref[...].astype(jnp.bfloat16),
                             (((1,), (1,)), ((), ())),
                             preferred_element_type=jnp.float32)
    m = jnp.max(logits, axis=1, keepdims=True)
    ex = jnp.exp(logits - m)
    probs = ex / jnp.sum(ex, axis=1, keepdims=True)

    ie = lax.broadcasted_iota(jnp.int32, (TB, E), 1)
    p1 = jnp.max(probs, axis=1)
    e1 = jnp.min(jnp.where(probs == p1[:, None], ie, E), axis=1)
    probs_m = jnp.where(ie == e1[:, None], -1.0, probs)
    p2 = jnp.max(probs_m, axis=1)
    e2 = jnp.min(jnp.where(probs_m == p2[:, None], ie, E), axis=1)

    oh1 = (ie == e1[:, None]).astype(jnp.float32)
    oh2 = (ie == e2[:, None]).astype(jnp.float32)
    a = oh1 + oh2
    ri = lax.broadcasted_iota(jnp.int32, (TB, TB), 0)
    ci = lax.broadcasted_iota(jnp.int32, (TB, TB), 1)
    tri = (ci < ri).astype(jnp.float32)
    prefix = lax.dot_general(tri, a, (((1,), (0,)), ((), ())),
                             precision=lax.Precision.HIGHEST,
                             preferred_element_type=jnp.float32)
    cp = carry_ref[...] + prefix
    pos1 = jnp.sum(cp * oh1, axis=1).astype(jnp.int32) + 1
    pos2 = jnp.sum(cp * oh2, axis=1).astype(jnp.int32) + 1
    carry_ref[...] += jnp.sum(a, axis=0, keepdims=True)

    slot1 = e1 * EROWS + b * C + pos1 - 1
    slot2 = e2 * EROWS + b * C + pos2 - 1
    keep1 = pos1 <= C
    keep2 = pos2 <= C
    d1_ref[0, 0, :] = jnp.where(keep1, slot1, SENT)
    d2_ref[0, 0, :] = jnp.where(keep2, slot2, SENT)
    g1_ref[0, 0, :] = jnp.where(keep1, slot1, ZROW)
    g2_ref[0, 0, :] = jnp.where(keep2, slot2, ZROW)
    p1_ref[0, 0, :] = p1
    p2_ref[0, 0, :] = p2


def _routing(x, gate_w):
    outs = pl.pallas_call(
        _routing_body,
        grid=(NBLK,),
        in_specs=[pl.BlockSpec((TB, H), lambda i: (i, 0)),
                  pl.BlockSpec((E, H), lambda i: (0, 0))],
        out_specs=[pl.BlockSpec((1, 1, TB), lambda i: (i, 0, 0))] * 6,
        out_shape=[jax.ShapeDtypeStruct((NBLK, 1, TB), jnp.int32)] * 4
        + [jax.ShapeDtypeStruct((NBLK, 1, TB), jnp.float32)] * 2,
        scratch_shapes=[pltpu.VMEM((1, E), jnp.float32)],
    )(x, gate_w)
    return [o.reshape(N) for o in outs]


def _sc_maps(d1, d2, p1, p2):
    @functools.partial(
        pl.kernel,
        out_type=(jax.ShapeDtypeStruct((R_TOT,), jnp.int32),
                  jax.ShapeDtypeStruct((R_TOT,), jnp.float32)),
        mesh=_sc_mesh(),
        compiler_params=_SC_PARAMS,
        scratch_types=[pltpu.VMEM((N,), jnp.int32),
                       pltpu.VMEM((N,), jnp.int32),
                       pltpu.VMEM((N,), jnp.float32),
                       pltpu.VMEM((N,), jnp.float32),
                       pltpu.VMEM((R_TOT,), jnp.int32),
                       pltpu.VMEM((R_TOT,), jnp.float32)],
    )
    def k(d1_hbm, d2_hbm, p1_hbm, p2_hbm, smap_hbm, wmap_hbm,
          d1_v, d2_v, p1_v, p2_v, smap_v, wmap_v):
        wid = lax.axis_index("s") * 2 + lax.axis_index("c")

        @pl.when(wid == 0)
        def _():
            pltpu.sync_copy(d1_hbm, d1_v)
            pltpu.sync_copy(d2_hbm, d2_v)
            pltpu.sync_copy(p1_hbm, p1_v)
            pltpu.sync_copy(p2_hbm, p2_v)

            def init(i, c):
                sl = pl.ds(i * 16, 16)
                smap_v[sl] = jnp.zeros((16,), jnp.int32)
                wmap_v[sl] = jnp.zeros((16,), jnp.float32)
                return c
            lax.fori_loop(0, R_TOT // 16, init, 0)

            def body(i, c):
                sl = pl.ds(i * 16, 16)
                tok = lax.iota(jnp.int32, 16) + i * 16
                d1c = d1_v[sl]
                plsc.store_scatter(smap_v, [d1c], tok, mask=d1c != SENT)
                plsc.store_scatter(wmap_v, [d1c], p1_v[sl], mask=d1c != SENT)
                d2c = d2_v[sl]
                plsc.store_scatter(smap_v, [d2c], tok, mask=d2c != SENT)
                plsc.store_scatter(wmap_v, [d2c], p2_v[sl], mask=d2c != SENT)
                return c
            lax.fori_loop(0, N // 16, body, 0)
            pltpu.sync_copy(smap_v, smap_hbm)
            pltpu.sync_copy(wmap_v, wmap_hbm)

    return k(d1, d2, p1, p2)


def _sc_dispatch(x, smap):
    chunks = [(o, 16) for o in range(0, 320, 16)] + [(320, 8)]
    nch = len(chunks)
    nbuf = 4

    @functools.partial(
        pl.kernel,
        out_type=jax.ShapeDtypeStruct((R_TOT, H), jnp.float32),
        mesh=_sc_mesh(),
        compiler_params=_SC_PARAMS,
        scratch_types=[pltpu.VMEM((RPW,), jnp.int32)]
        + [pltpu.VMEM((16, H), jnp.float32)] * 4
        + [pltpu.SemaphoreType.DMA] * 8,
    )
    def k(x_hbm, smap_hbm, disp_hbm, idx_v, b0, b1, b2, b3,
          g0, g1s, g2s, g3s, s0, s1, s2, s3):
        wid = lax.axis_index("s") * 2 + lax.axis_index("c")
        base = wid * RPW
        bufs = (b0, b1, b2, b3)
        gsems = (g0, g1s, g2s, g3s)
        ssems = (s0, s1, s2, s3)
        pltpu.sync_copy(smap_hbm.at[pl.ds(base, RPW)], idx_v)

        def gather(c):
            off, sz = chunks[c]
            return pltpu.make_async_copy(
                x_hbm.at[idx_v.at[pl.ds(off, sz)]],
                bufs[c % nbuf].at[pl.ds(0, sz)], gsems[c % nbuf])

        def store(c):
            off, sz = chunks[c]
            return pltpu.make_async_copy(
                bufs[c % nbuf].at[pl.ds(0, sz)],
                disp_hbm.at[pl.ds(base + off, sz)], ssems[c % nbuf])

        gd = {}
        for c in range(nbuf):
            gd[c] = gather(c)
            gd[c].start()
        sd = {}
        for c in range(nch):
            if c >= 1 and c + nbuf - 1 < nch:
                sd[c - 1].wait()
                gd[c + nbuf - 1] = gather(c + nbuf - 1)
                gd[c + nbuf - 1].start()
            gd[c].wait()
            sd[c] = store(c)
            sd[c].start()
        for c in range(max(0, nch - nbuf), nch):
            sd[c].wait()

    return k(x, smap)


def _mlp_body(x_ref, w1_ref, w3_ref, w2_ref, wt_ref, out_ref):
    j = pl.program_id(1)
    x = x_ref[...].astype(jnp.bfloat16)
    h1 = jnp.dot(x, w1_ref[0].astype(jnp.bfloat16),
                 preferred_element_type=jnp.float32)
    h3 = jnp.dot(x, w3_ref[0].astype(jnp.bfloat16),
                 preferred_element_type=jnp.float32)
    hh = (h1 / (1.0 + jnp.exp(-h1))) * h3
    contrib = jnp.dot(hh.astype(jnp.bfloat16), w2_ref[0].astype(jnp.bfloat16),
                      preferred_element_type=jnp.float32)

    @pl.when(j == 0)
    def _():
        out_ref[...] = contrib

    @pl.when(jnp.logical_and(j > 0, j < NJ - 1))
    def _():
        out_ref[...] += contrib

    @pl.when(j == NJ - 1)
    def _():
        out_ref[...] = (out_ref[...] + contrib) * wt_ref[...]


def _mlp(disp, w1, w3, w2, wt):
    return pl.pallas_call(
        _mlp_body,
        grid=(E, NJ),
        in_specs=[pl.BlockSpec((EROWS, H), lambda e, j: (e, 0)),
                  pl.BlockSpec((1, H, FB), lambda e, j: (e, 0, j)),
                  pl.BlockSpec((1, H, FB), lambda e, j: (e, 0, j)),
                  pl.BlockSpec((1, FB, H), lambda e, j: (e, j, 0)),
                  pl.BlockSpec((EROWS, 1), lambda e, j: (e, 0))],
        out_specs=pl.BlockSpec((EROWS, H), lambda e, j: (e, 0)),
        out_shape=jax.ShapeDtypeStruct((R_TOT, H), jnp.float32),
    )(disp, w1, w3, w2, wt)


def _sc_combine(eo, g1, g2):
    @functools.partial(
        pl.kernel,
        out_type=jax.ShapeDtypeStruct((N, H), jnp.float32),
        mesh=_sc_mesh(),
        compiler_params=_SC_PARAMS,
        scratch_types=[pltpu.VMEM((TPW,), jnp.int32),
                       pltpu.VMEM((TPW,), jnp.int32),
                       pltpu.VMEM((CH, H), jnp.float32),
                       pltpu.VMEM((CH, H), jnp.float32),
                       pltpu.VMEM((CH, H), jnp.float32),
                       pltpu.VMEM((CH, H), jnp.float32),
                       pltpu.SemaphoreType.DMA,
                       pltpu.SemaphoreType.DMA,
                       pltpu.SemaphoreType.DMA,
                       pltpu.SemaphoreType.DMA,
                       pltpu.SemaphoreType.DMA,
                       pltpu.SemaphoreType.DMA],
    )
    def k(eo_hbm, g1_hbm, g2_hbm, out_hbm, g1_v, g2_v,
          a0, a1, c0, c1, sa0, sa1, sc0, sc1, so0, so1):
        wid = lax.axis_index("s") * 2 + lax.axis_index("c")
        base = wid * TPW
        abuf = (a0, a1)
        bbuf = (c0, c1)
        asem = (sa0, sa1)
        bsem = (sc0, sc1)
        osem = (so0, so1)
        nch = TPW // CH
        pltpu.sync_copy(g1_hbm.at[pl.ds(base, TPW)], g1_v)
        pltpu.sync_copy(g2_hbm.at[pl.ds(base, TPW)], g2_v)

        def gathers(c):
            p = c % 2
            d1 = pltpu.make_async_copy(
                eo_hbm.at[g1_v.at[pl.ds(c * CH, CH)]], abuf[p], asem[p])
            d2 = pltpu.make_async_copy(
                eo_hbm.at[g2_v.at[pl.ds(c * CH, CH)]], bbuf[p], bsem[p])
            d1.start()
            d2.start()
            return d1, d2

        def outstore(c):
            p = c % 2
            return pltpu.make_async_copy(
                abuf[p], out_hbm.at[pl.ds(base + c * CH, CH)], osem[p])

        gd = {0: gathers(0)}
        od = {}
        for c in range(nch):
            if c + 1 < nch:
                if c >= 1:
                    od[c - 1].wait()
                gd[c + 1] = gathers(c + 1)
            d1, d2 = gd[c]
            d1.wait()
            d2.wait()
            p = c % 2

            def add(i, cc):
                r = i // 8
                col0 = lax.rem(i, 8) * 128
                for u in range(8):
                    sl = pl.ds(col0 + u * 16, 16)
                    abuf[p][r, sl] = abuf[p][r, sl] + bbuf[p][r, sl]
                return cc
            lax.fori_loop(0, CH * 8, add, 0)
            od[c] = outstore(c)
            od[c].start()
        od[nch - 2].wait()
        od[nch - 1].wait()

    return k(eo, g1, g2)


def kernel(hidden_states, gate_w, w1, w2, w3):
    x = hidden_states.reshape(N, H)
    d1, d2, g1, g2, p1, p2 = _routing(x, gate_w)
    smap, wmap = _sc_maps(d1, d2, p1, p2)
    disp = _sc_dispatch(x, smap)
    eo = _mlp(disp, w1, w3, w2, wmap.reshape(R_TOT, 1))
    out = _sc_combine(eo, g1, g2)
    return out.reshape(B, S, H)
